# baseline (speedup 1.0000x reference)
import jax
import jax.numpy as jnp
from jax.experimental import pallas as pl
from jax.experimental.pallas import tpu as pltpu
from jax.experimental.pallas import tpu_sc as plsc

N_NODES = 10000
N_PAD = 10240
N_EDGES = 320000
D_IN = 128
D_HIDDEN = 128
D_OUT = 40
D_OUT_PAD = 48
WD = 0.0005

NUM_SC = 2
NUM_SUBCORES = 16
LANES = 16


def _spmm_partials(xh, src, dst, w, d, window, chunks, depth=3):
    per_worker = N_EDGES // (NUM_SC * NUM_SUBCORES)
    n_win = per_worker // window
    assert sum(chunks) == n_win
    ch_max = max(chunks)
    rows_per_sub = N_PAD // NUM_SUBCORES

    mesh = plsc.VectorSubcoreMesh(core_axis_name="c", subcore_axis_name="s")

    def body(xh_hbm, src_hbm, dst_hbm, w_hbm, zero_hbm, out_hbm, acc, *rest):
        rows_list = rest[:depth]
        sbig, dbig, wbig = rest[depth:depth + 3]
        sgs = rest[depth + 3:2 * depth + 3]
        sss = rest[2 * depth + 3:3 * depth + 3]
        sz, szz = rest[3 * depth + 3:]
        c = jax.lax.axis_index("c")
        s = jax.lax.axis_index("s")
        r0 = s * rows_per_sub
        pltpu.async_copy(zero_hbm.at[pl.ds(r0, rows_per_sub)],
                         acc.at[pl.ds(r0, rows_per_sub)], szz)

        win0 = (c * NUM_SUBCORES + s) * n_win
        bufs = list(zip(rows_list, sgs, sss))

        def scale_rows(rows, j):
            @pl.loop(0, window // LANES)
            def _g(g):
                wv = wbig[j, pl.ds(g * LANES, LANES)]
                for e in range(LANES):
                    i = g * LANES + e
                    wi = wv[e]
                    for q in range(d // LANES):
                        sl = pl.ds(q * LANES, LANES)
                        rows[i, sl] = rows[i, sl] * wi

        def gather_wait(rows, sg):
            pltpu.make_async_copy(xh_hbm.at[sbig.at[0]], rows, sg).wait()

        def scatter_wait(rows, ss):
            pltpu.make_async_copy(zero_hbm.at[pl.ds(0, window)], rows, ss).wait()

        def win_iter(j, b, ch_sz):
            rows_j, sg_j, ss_j = bufs[b]
            rows_p, _, ss_p = bufs[(b - 1) % depth]
            rows_n, sg_n, _ = bufs[(b + depth - 1) % depth]
            gather_wait(rows_j, sg_j)
            scale_rows(rows_j, j)
            pltpu.async_copy(rows_j, acc.at[dbig.at[j]], ss_j, add=True)

            @pl.when(j > 0)
            def _():
                scatter_wait(rows_p, ss_p)

            @pl.when(j + depth - 1 < ch_sz)
            def _():
                pltpu.async_copy(xh_hbm.at[sbig.at[j + depth - 1]],
                                 rows_n, sg_n)

        ch_base = win0
        for ci, ch_sz in enumerate(chunks):
            pltpu.async_copy(src_hbm.at[pl.ds(ch_base, ch_sz)],
                             sbig.at[pl.ds(0, ch_sz)], sz).wait()
            pltpu.async_copy(dst_hbm.at[pl.ds(ch_base, ch_sz)],
                             dbig.at[pl.ds(0, ch_sz)], sz).wait()
            pltpu.async_copy(w_hbm.at[pl.ds(ch_base, ch_sz)],
                             wbig.at[pl.ds(0, ch_sz)], sz).wait()

            for pj in range(min(depth - 1, ch_sz)):
                pltpu.async_copy(xh_hbm.at[sbig.at[pj]],
                                 bufs[pj][0], bufs[pj][1])

            if ci == 0:
                pltpu.make_async_copy(
                    zero_hbm.at[pl.ds(r0, rows_per_sub)],
                    acc.at[pl.ds(r0, rows_per_sub)], szz).wait()
                plsc.subcore_barrier()

            @pl.loop(0, ch_sz)
            def _win(j):
                m = j % depth
                for b in range(depth):
                    @pl.when(m == b)
                    def _(b=b):
                        win_iter(j, b, ch_sz)

            fb = (ch_sz - 1) % depth
            scatter_wait(bufs[fb][0], bufs[fb][2])
            ch_base += ch_sz

        plsc.subcore_barrier()
        pltpu.async_copy(acc.at[pl.ds(r0, rows_per_sub)],
                         out_hbm.at[c].at[pl.ds(r0, rows_per_sub)], sz).wait()

    kfn = pl.kernel(
        body,
        out_type=jax.ShapeDtypeStruct((NUM_SC, N_PAD, d), jnp.float32),
        mesh=mesh,
        compiler_params=pltpu.CompilerParams(use_tc_tiling_on_sc=False),
        scratch_types=(
            [pltpu.VMEM_SHARED((N_PAD, d), jnp.float32)]
            + [pltpu.VMEM((window, d), jnp.float32)] * depth
            + [pltpu.VMEM((ch_max, window), jnp.int32),
               pltpu.VMEM((ch_max, window), jnp.int32),
               pltpu.VMEM((ch_max, window), jnp.float32)]
            + [pltpu.SemaphoreType.DMA] * (2 * depth + 2)
        ),
    )
    zeros = jnp.zeros((N_PAD, d), jnp.float32)
    total_win = N_EDGES // window
    return kfn(xh, src.reshape(total_win, window), dst.reshape(total_win, window),
               w.reshape(total_win, window), zeros)


def _hidden_body(p_ref, w0_ref, w1_ref, o_ref):
    t = (p_ref[0] + p_ref[1]).astype(jnp.bfloat16)
    t = jnp.dot(t, w0_ref[...].astype(jnp.bfloat16),
                preferred_element_type=jnp.float32)
    h = jnp.maximum(t, 0.0).astype(jnp.bfloat16)
    o_ref[...] = jnp.dot(h, w1_ref[...].astype(jnp.bfloat16),
                         preferred_element_type=jnp.float32)


def _loss_body(p_ref, lab_ref, m_ref, w0_ref, loss_ref, acc_ref):
    out = (p_ref[0] + p_ref[1])[:N_NODES]
    col = jax.lax.broadcasted_iota(jnp.int32, (N_NODES, D_OUT_PAD), 1)
    valid = col < D_OUT
    z = jnp.where(valid, out, jnp.float32(-1e30))
    mx = jnp.max(z, axis=1, keepdims=True)
    ez = jnp.where(valid, jnp.exp(z - mx), 0.0)
    lse = jnp.log(jnp.sum(ez, axis=1, keepdims=True)) + mx
    lab = lab_ref[...]
    onehot = (col == lab) & valid
    pick = jnp.sum(jnp.where(onehot, z, 0.0), axis=1, keepdims=True)
    ce = lse - pick
    m = m_ref[...].astype(jnp.float32)
    sum_m = jnp.sum(m)
    correct = (pick >= mx).astype(jnp.float32)
    l2 = 0.5 * WD * jnp.sum(w0_ref[...] * w0_ref[...])
    loss_ref[0, 0] = l2 + jnp.sum(ce * m) / sum_m
    acc_ref[0, 0] = jnp.sum(correct * m) / sum_m


def kernel(x, label, mask, edge_index, edge_weight, W0, W1):
    src = edge_index[0]
    dst = edge_index[1]

    p1 = _spmm_partials(x, src, dst, edge_weight, D_IN, window=80,
                        chunks=[63, 62], depth=3)

    w1p = jnp.pad(W1, ((0, 0), (0, D_OUT_PAD - D_OUT)))
    y2 = pl.pallas_call(
        _hidden_body,
        out_shape=jax.ShapeDtypeStruct((N_PAD, D_OUT_PAD), jnp.float32),
    )(p1, W0, w1p)

    p2 = _spmm_partials(y2, src, dst, edge_weight, D_OUT_PAD, window=400,
                        chunks=[25], depth=3)

    lab2 = label.reshape(N_NODES, 1)
    m2 = mask.reshape(N_NODES, 1)
    loss, acc = pl.pallas_call(
        _loss_body,
        out_shape=[
            jax.ShapeDtypeStruct((1, 1), jnp.float32),
            jax.ShapeDtypeStruct((1, 1), jnp.float32),
        ],
        out_specs=[
            pl.BlockSpec(memory_space=pltpu.SMEM),
            pl.BlockSpec(memory_space=pltpu.SMEM),
        ],
    )(p2, lab2, m2, W0)
    return (loss.reshape(()), acc.reshape(()))


# speedup vs baseline: 1.0021x; 1.0021x over previous
import jax
import jax.numpy as jnp
from jax.experimental import pallas as pl
from jax.experimental.pallas import tpu as pltpu
from jax.experimental.pallas import tpu_sc as plsc

N_NODES = 10000
N_PAD = 10240
N_EDGES = 320000
D_IN = 128
D_HIDDEN = 128
D_OUT = 40
D_OUT_PAD = 48
WD = 0.0005

NUM_SC = 2
NUM_SUBCORES = 16
LANES = 16


def _spmm_partials(xh, src, dst, w, d, window, chunks, depth=3):
    per_worker = N_EDGES // (NUM_SC * NUM_SUBCORES)
    n_win = per_worker // window
    assert sum(chunks) == n_win
    ch_max = max(chunks)
    rows_per_sub = N_PAD // NUM_SUBCORES

    mesh = plsc.VectorSubcoreMesh(core_axis_name="c", subcore_axis_name="s")

    def body(xh_hbm, src_hbm, dst_hbm, w_hbm, zero_hbm, out_hbm, acc, *rest):
        rows_list = rest[:depth]
        sbig, dbig, wbig = rest[depth:depth + 3]
        sgs = rest[depth + 3:2 * depth + 3]
        sss = rest[2 * depth + 3:3 * depth + 3]
        sz, szz = rest[3 * depth + 3:]
        c = jax.lax.axis_index("c")
        s = jax.lax.axis_index("s")
        r0 = s * rows_per_sub
        pltpu.async_copy(zero_hbm.at[pl.ds(r0, rows_per_sub)],
                         acc.at[pl.ds(r0, rows_per_sub)], szz)

        win0 = (c * NUM_SUBCORES + s) * n_win
        bufs = list(zip(rows_list, sgs, sss))

        def scale_rows(rows, j):
            @pl.loop(0, window // LANES)
            def _g(g):
                wv = wbig[j, pl.ds(g * LANES, LANES)]
                for e in range(LANES):
                    i = g * LANES + e
                    wi = wv[e]
                    for q in range(d // LANES):
                        sl = pl.ds(q * LANES, LANES)
                        rows[i, sl] = rows[i, sl] * wi

        def gather_wait(rows, sg):
            pltpu.make_async_copy(xh_hbm.at[sbig.at[0]], rows, sg).wait()

        def scatter_wait(rows, ss):
            pltpu.make_async_copy(zero_hbm.at[pl.ds(0, window)], rows, ss).wait()

        def win_iter(j, b, ch_sz):
            rows_j, sg_j, ss_j = bufs[b]
            rows_p, _, ss_p = bufs[(b - 1) % depth]
            rows_n, sg_n, _ = bufs[(b + depth - 1) % depth]
            gather_wait(rows_j, sg_j)
            scale_rows(rows_j, j)
            pltpu.async_copy(rows_j, acc.at[dbig.at[j]], ss_j, add=True)

            @pl.when(j > 0)
            def _():
                scatter_wait(rows_p, ss_p)

            @pl.when(j + depth - 1 < ch_sz)
            def _():
                pltpu.async_copy(xh_hbm.at[sbig.at[j + depth - 1]],
                                 rows_n, sg_n)

        ch_base = win0
        for ci, ch_sz in enumerate(chunks):
            pltpu.async_copy(src_hbm.at[pl.ds(ch_base, ch_sz)],
                             sbig.at[pl.ds(0, ch_sz)], sz).wait()
            pltpu.async_copy(dst_hbm.at[pl.ds(ch_base, ch_sz)],
                             dbig.at[pl.ds(0, ch_sz)], sz).wait()
            pltpu.async_copy(w_hbm.at[pl.ds(ch_base, ch_sz)],
                             wbig.at[pl.ds(0, ch_sz)], sz).wait()

            for pj in range(min(depth - 1, ch_sz)):
                pltpu.async_copy(xh_hbm.at[sbig.at[pj]],
                                 bufs[pj][0], bufs[pj][1])

            if ci == 0:
                pltpu.make_async_copy(
                    zero_hbm.at[pl.ds(r0, rows_per_sub)],
                    acc.at[pl.ds(r0, rows_per_sub)], szz).wait()
                plsc.subcore_barrier()

            @pl.loop(0, ch_sz)
            def _win(j):
                m = j % depth
                for b in range(depth):
                    @pl.when(m == b)
                    def _(b=b):
                        win_iter(j, b, ch_sz)

            fb = (ch_sz - 1) % depth
            scatter_wait(bufs[fb][0], bufs[fb][2])
            ch_base += ch_sz

        plsc.subcore_barrier()
        pltpu.async_copy(acc.at[pl.ds(r0, rows_per_sub)],
                         out_hbm.at[c].at[pl.ds(r0, rows_per_sub)], sz).wait()

    kfn = pl.kernel(
        body,
        out_type=jax.ShapeDtypeStruct((NUM_SC, N_PAD, d), jnp.float32),
        mesh=mesh,
        compiler_params=pltpu.CompilerParams(use_tc_tiling_on_sc=False),
        scratch_types=(
            [pltpu.VMEM_SHARED((N_PAD, d), jnp.float32)]
            + [pltpu.VMEM((window, d), jnp.float32)] * depth
            + [pltpu.VMEM((ch_max, window), jnp.int32),
               pltpu.VMEM((ch_max, window), jnp.int32),
               pltpu.VMEM((ch_max, window), jnp.float32)]
            + [pltpu.SemaphoreType.DMA] * (2 * depth + 2)
        ),
    )
    zeros = jnp.zeros((N_PAD, d), jnp.float32)
    total_win = N_EDGES // window
    return kfn(xh, src.reshape(total_win, window), dst.reshape(total_win, window),
               w.reshape(total_win, window), zeros)


def _hidden_body(p_ref, w0_ref, w1_ref, o_ref):
    t = p_ref[0] + p_ref[1]
    t = jnp.dot(t, w0_ref[...], preferred_element_type=jnp.float32)
    h = jnp.maximum(t, 0.0)
    o_ref[...] = jnp.dot(h, w1_ref[...], preferred_element_type=jnp.float32)


def _loss_body(p_ref, lab_ref, m_ref, w0_ref, loss_ref, acc_ref):
    out = (p_ref[0] + p_ref[1])[:N_NODES]
    col = jax.lax.broadcasted_iota(jnp.int32, (N_NODES, D_OUT_PAD), 1)
    valid = col < D_OUT
    z = jnp.where(valid, out, jnp.float32(-1e30))
    mx = jnp.max(z, axis=1, keepdims=True)
    ez = jnp.where(valid, jnp.exp(z - mx), 0.0)
    lse = jnp.log(jnp.sum(ez, axis=1, keepdims=True)) + mx
    lab = lab_ref[...]
    onehot = (col == lab) & valid
    pick = jnp.sum(jnp.where(onehot, z, 0.0), axis=1, keepdims=True)
    ce = lse - pick
    m = m_ref[...].astype(jnp.float32)
    sum_m = jnp.sum(m)
    correct = (pick >= mx).astype(jnp.float32)
    l2 = 0.5 * WD * jnp.sum(w0_ref[...] * w0_ref[...])
    loss_ref[0, 0] = l2 + jnp.sum(ce * m) / sum_m
    acc_ref[0, 0] = jnp.sum(correct * m) / sum_m


def kernel(x, label, mask, edge_index, edge_weight, W0, W1):
    src = edge_index[0]
    dst = edge_index[1]

    p1 = _spmm_partials(x, src, dst, edge_weight, D_IN, window=80,
                        chunks=[63, 62], depth=3)

    w1p = jnp.pad(W1, ((0, 0), (0, D_OUT_PAD - D_OUT)))
    y2 = pl.pallas_call(
        _hidden_body,
        out_shape=jax.ShapeDtypeStruct((N_PAD, D_OUT_PAD), jnp.float32),
    )(p1, W0, w1p)

    p2 = _spmm_partials(y2, src, dst, edge_weight, D_OUT_PAD, window=400,
                        chunks=[25], depth=3)

    lab2 = label.reshape(N_NODES, 1)
    m2 = mask.reshape(N_NODES, 1)
    loss, acc = pl.pallas_call(
        _loss_body,
        out_shape=[
            jax.ShapeDtypeStruct((1, 1), jnp.float32),
            jax.ShapeDtypeStruct((1, 1), jnp.float32),
        ],
        out_specs=[
            pl.BlockSpec(memory_space=pltpu.SMEM),
            pl.BlockSpec(memory_space=pltpu.SMEM),
        ],
    )(p2, lab2, m2, W0)
    return (loss.reshape(()), acc.reshape(()))
